# baseline (speedup 1.0000x reference)
import jax
import jax.numpy as jnp
from jax.experimental import pallas as pl

N = 10000
E = 320000
HEADS = 5
C_HEAD = 32
HID = 160
G = 8


def _ln(x, g, b):
    m = jnp.mean(x, axis=-1, keepdims=True)
    v = jnp.var(x, axis=-1, keepdims=True)
    return (x - m) / jnp.sqrt(v + 1e-5) * g + b


def _gat(x, s, d, W, a_src, a_dst, bias):
    h = (x @ W).reshape(N, HEADS, -1)
    a_s = jnp.sum(h * a_src[None], axis=-1)
    a_d = jnp.sum(h * a_dst[None], axis=-1)
    alpha = a_s[s] + a_d[d]
    alpha = jnp.where(alpha > 0, alpha, 0.2 * alpha)
    amax = jax.ops.segment_max(alpha, d, num_segments=N)
    amax = jnp.where(jnp.isfinite(amax), amax, 0.0)
    ex = jnp.exp(alpha - amax[d])
    den = jax.ops.segment_sum(ex, d, num_segments=N)
    att = ex / (den[d] + 1e-16)
    msg = h[s] * att[..., None]
    out = jax.ops.segment_sum(msg, d, num_segments=N)
    return out.reshape(N, HEADS * h.shape[-1]) + bias, att


def _copy_kernel(x_ref, o_ref):
    o_ref[...] = x_ref[...]


def kernel(x, edge_index, batch, W1, a_src1, a_dst1, b1, W2, a_src2, a_dst2, b2,
           g1, be1, g2, be2, Wm, bm, Wr1, br1, gr, ber, Wr2, br2):
    src = edge_index[0]
    dst = edge_index[1]
    loop = jnp.arange(N, dtype=src.dtype)
    s = jnp.concatenate([src, loop])
    d = jnp.concatenate([dst, loop])
    h1, att1 = _gat(x, s, d, W1, a_src1, a_dst1, b1)
    h1 = jnp.where(h1 > 0, h1, 0.01 * h1)
    h1 = _ln(h1, g1, be1)
    h2, att2 = _gat(h1, s, d, W2, a_src2, a_dst2, b2)
    h2 = jnp.where(h2 > 0, h2, 0.01 * h2)
    h2 = _ln(h2, g2, be2)
    xo = h2 @ Wm + bm
    xo = jnp.where(xo > 0, xo, 0.01 * xo)
    xo = pl.pallas_call(
        _copy_kernel,
        out_shape=jax.ShapeDtypeStruct(xo.shape, xo.dtype),
    )(xo)
    pooled_sum = jax.ops.segment_sum(xo, batch, num_segments=G)
    cnt = jax.ops.segment_sum(jnp.ones((N,), jnp.float32), batch, num_segments=G)
    pooled = pooled_sum / jnp.maximum(cnt, 1.0)[:, None]
    r = pooled @ Wr1 + br1
    r = _ln(r, gr, ber)
    r = jnp.maximum(r, 0.0)
    rec = r @ Wr2 + br2
    return (xo, rec, att1, att2)


# speedup vs baseline: 41.2869x; 41.2869x over previous
import functools

import numpy as np
import jax
import jax.numpy as jnp
from jax import lax
from jax.experimental import pallas as pl
from jax.experimental.pallas import tpu as pltpu
from jax.experimental.pallas import tpu_sc as plsc

N = 10000
E = 320000
HEADS = 5
C_HEAD = 32
HID = 160
G = 8
NC = 20

NWORK = 32
CHUNK = 128
E_TOT = E + N
EPW = 10368
EPAD = NWORK * EPW
NCHUNK = EPW // CHUNK
NPAD = 10240
ROWS_PT = NPAD // 16

_SC_PARAMS = pltpu.CompilerParams(
    needs_layout_passes=False, use_tc_tiling_on_sc=False)


def _ln(x, g, b):
    m = jnp.mean(x, axis=-1, keepdims=True)
    v = jnp.var(x, axis=-1, keepdims=True)
    return (x - m) / jnp.sqrt(v + 1e-5) * g + b


def _proj(h, m_ref, asd_ref, c_ref):
    asd = jnp.dot(h, m_ref[...], preferred_element_type=jnp.float32)
    asd_ref[...] = asd
    mx = jnp.max(asd[:, :2 * HEADS], axis=0)
    raw = mx[:HEADS] + mx[HEADS:]
    c = jnp.where(raw > 0, raw, 0.2 * raw)
    c_ref[...] = jnp.pad(c, (0, 11))


def _pre_body(x_ref, w_ref, m_ref, h_ref, asd_ref, c_ref):
    h = jnp.dot(x_ref[...], w_ref[...], preferred_element_type=jnp.float32)
    h_ref[...] = h
    _proj(h, m_ref, asd_ref, c_ref)


def _tc_pre(x, w, m):
    return pl.pallas_call(
        _pre_body,
        out_shape=(
            jax.ShapeDtypeStruct((N, HID), jnp.float32),
            jax.ShapeDtypeStruct((N, 16), jnp.float32),
            jax.ShapeDtypeStruct((16,), jnp.float32),
        ),
    )(x, w, m)


def _dinv_body(den_ref, o_ref):
    d = den_ref[0] + den_ref[1]
    o_ref[...] = 1.0 / jnp.maximum(d, 1e-30)


def _tc_dinv(den):
    return pl.pallas_call(
        _dinv_body,
        out_shape=jax.ShapeDtypeStruct((NPAD, 16), jnp.float32),
    )(den)


def _mid_body(agg_ref, b_ref, g_ref, be_ref, w_ref, m_ref,
              h2_ref, asd_ref, c_ref):
    o = agg_ref[0, :N] + agg_ref[1, :N] + b_ref[...]
    o = jnp.where(o > 0, o, 0.01 * o)
    h1 = _ln(o, g_ref[...], be_ref[...])
    h2 = jnp.dot(h1, w_ref[...], preferred_element_type=jnp.float32)
    h2_ref[...] = h2
    _proj(h2, m_ref, asd_ref, c_ref)


def _tc_mid(agg, b, g, be, w, m):
    return pl.pallas_call(
        _mid_body,
        out_shape=(
            jax.ShapeDtypeStruct((N, HID), jnp.float32),
            jax.ShapeDtypeStruct((N, 16), jnp.float32),
            jax.ShapeDtypeStruct((16,), jnp.float32),
        ),
    )(agg, b, g, be, w, m)


def _final_body(agg_ref, b_ref, g_ref, be_ref, wm_ref, bm_ref, oh_ref,
                wr1_ref, br1_ref, gr_ref, ber_ref, wr2_ref, br2_ref,
                xo_ref, rec_ref):
    o = agg_ref[0, :N] + agg_ref[1, :N] + b_ref[...]
    o = jnp.where(o > 0, o, 0.01 * o)
    h2 = _ln(o, g_ref[...], be_ref[...])
    xo = jnp.dot(h2, wm_ref[...], preferred_element_type=jnp.float32)
    xo = xo + bm_ref[...]
    xo = jnp.where(xo > 0, xo, 0.01 * xo)
    xo_ref[...] = xo
    oh = oh_ref[...]
    pooled = jnp.dot(oh, xo, preferred_element_type=jnp.float32)
    cnt = jnp.sum(oh, axis=1)
    pooled = pooled / jnp.maximum(cnt, 1.0)[:, None]
    r = jnp.dot(pooled, wr1_ref[...], preferred_element_type=jnp.float32)
    r = r + br1_ref[...]
    r = _ln(r, gr_ref[...], ber_ref[...])
    r = jnp.maximum(r, 0.0)
    rec = jnp.dot(r, wr2_ref[...], preferred_element_type=jnp.float32)
    rec_ref[...] = rec + br2_ref[...]


def _tc_final(agg, b, g, be, wm, bm, oh, wr1, br1, gr, ber, wr2, br2):
    return pl.pallas_call(
        _final_body,
        out_shape=(
            jax.ShapeDtypeStruct((N, HID), jnp.float32),
            jax.ShapeDtypeStruct((G, NC), jnp.float32),
        ),
    )(agg, b, g, be, wm, bm, oh, wr1, br1, gr, ber, wr2, br2)


def _sca_body(s_hbm, d_hbm, asd_hbm, c_hbm, ex_hbm, den_hbm,
              cbuf, sbuf, dbuf, exbuf, arows, brows, sem, sem2, den_sp):
    cid = lax.axis_index("c")
    sid = lax.axis_index("s")
    wid = sid * 2 + cid
    base = wid * EPW

    pltpu.sync_copy(c_hbm, cbuf)
    cvec = cbuf[...]

    def zex(i, _):
        exbuf[i, :] = jnp.zeros((16,), jnp.float32)
        return 0
    lax.fori_loop(0, CHUNK, zex, 0)
    for b5 in range(ROWS_PT // CHUNK):
        pltpu.sync_copy(exbuf,
                        den_sp.at[pl.ds(sid * ROWS_PT + b5 * CHUNK, CHUNK)])
    plsc.subcore_barrier()

    iota = lax.broadcasted_iota(jnp.int32, (16,), 0)

    def chunk_body(ci, _):
        off = base + ci * CHUNK
        pltpu.sync_copy(s_hbm.at[pl.ds(off, CHUNK)], sbuf)
        pltpu.sync_copy(d_hbm.at[pl.ds(off, CHUNK)], dbuf)
        cp_a = pltpu.async_copy(asd_hbm.at[sbuf], arows, sem)
        cp_b = pltpu.async_copy(asd_hbm.at[dbuf], brows, sem2)
        cp_a.wait()
        cp_b.wait()
        for j in range(CHUNK // 16):
            eid = off + j * 16 + iota
            valid = eid < E_TOT
            row = jnp.full((16,), j * 16, jnp.int32) + iota
            for hd in range(HEADS):
                hvec = jnp.full((16,), hd, jnp.int32)
                a = plsc.load_gather(arows, [row, hvec])
                b = plsc.load_gather(brows, [row, hvec + HEADS])
                al = a + b
                al = jnp.where(al > 0, al, 0.2 * al)
                ex = jnp.exp(al - cvec[hd])
                ex = jnp.where(valid, ex, 0.0)
                plsc.store_scatter(exbuf, [row, hvec], ex)
        pltpu.sync_copy(exbuf, ex_hbm.at[pl.ds(off, CHUNK)])
        pltpu.sync_copy(exbuf, den_sp.at[dbuf], add=True)
        return 0

    lax.fori_loop(0, NCHUNK, chunk_body, 0)

    plsc.subcore_barrier()
    pltpu.sync_copy(den_sp.at[pl.ds(sid * ROWS_PT, ROWS_PT)],
                    den_hbm.at[cid, pl.ds(sid * ROWS_PT, ROWS_PT)])


@functools.partial(
    pl.kernel,
    out_type=(
        jax.ShapeDtypeStruct((EPAD, 16), jnp.float32),
        jax.ShapeDtypeStruct((2, NPAD, 16), jnp.float32),
    ),
    mesh=plsc.VectorSubcoreMesh(core_axis_name="c", subcore_axis_name="s"),
    compiler_params=_SC_PARAMS,
    scratch_types=[
        pltpu.VMEM((16,), jnp.float32),
        pltpu.VMEM((CHUNK,), jnp.int32),
        pltpu.VMEM((CHUNK,), jnp.int32),
        pltpu.VMEM((CHUNK, 16), jnp.float32),
        pltpu.VMEM((CHUNK, 16), jnp.float32),
        pltpu.VMEM((CHUNK, 16), jnp.float32),
        pltpu.SemaphoreType.DMA,
        pltpu.SemaphoreType.DMA,
        pltpu.VMEM_SHARED((NPAD, 16), jnp.float32),
    ],
)
def _sc_a(s_hbm, d_hbm, asd_hbm, c_hbm, ex_hbm, den_hbm,
          cbuf, sbuf, dbuf, exbuf, arows, brows, sem, sem2, den_sp):
    _sca_body(s_hbm, d_hbm, asd_hbm, c_hbm, ex_hbm, den_hbm,
              cbuf, sbuf, dbuf, exbuf, arows, brows, sem, sem2, den_sp)


def _scc_body(s_hbm, d_hbm, ex_hbm, dinv_hbm, h_hbm, att_hbm, out_hbm,
              sbuf, dbuf, exbuf, drows, hrows, sem, sem2, out_sp):
    cid = lax.axis_index("c")
    sid = lax.axis_index("s")
    wid = sid * 2 + cid
    base = wid * EPW

    def zrow(i, _):
        for k in range(HID // 16):
            hrows[i, pl.ds(k * 16, 16)] = jnp.zeros((16,), jnp.float32)
        return 0
    lax.fori_loop(0, CHUNK, zrow, 0)
    for b5 in range(ROWS_PT // CHUNK):
        pltpu.sync_copy(hrows,
                        out_sp.at[pl.ds(sid * ROWS_PT + b5 * CHUNK, CHUNK)])
    plsc.subcore_barrier()

    iota = lax.broadcasted_iota(jnp.int32, (16,), 0)

    def chunk_body(ci, _):
        off = base + ci * CHUNK
        pltpu.sync_copy(s_hbm.at[pl.ds(off, CHUNK)], sbuf)
        pltpu.sync_copy(d_hbm.at[pl.ds(off, CHUNK)], dbuf)
        pltpu.sync_copy(ex_hbm.at[pl.ds(off, CHUNK)], exbuf)
        cp_h = pltpu.async_copy(h_hbm.at[sbuf], hrows, sem)
        cp_d = pltpu.async_copy(dinv_hbm.at[dbuf], drows, sem2)
        cp_h.wait()
        cp_d.wait()
        for j in range(CHUNK // 16):
            row = jnp.full((16,), j * 16, jnp.int32) + iota
            for hd in range(HEADS):
                hvec = jnp.full((16,), hd, jnp.int32)
                di = plsc.load_gather(drows, [row, hvec])
                ev = plsc.load_gather(exbuf, [row, hvec])
                plsc.store_scatter(exbuf, [row, hvec], ev * di)

        def scale(e, _):
            av = exbuf[e]
            for k in range(HID // 16):
                a = av[k // 2]
                sl = pl.ds(k * 16, 16)
                hrows[e, sl] = hrows[e, sl] * a
            return 0
        lax.fori_loop(0, CHUNK, scale, 0)

        pltpu.sync_copy(exbuf, att_hbm.at[pl.ds(off, CHUNK)])
        pltpu.sync_copy(hrows, out_sp.at[dbuf], add=True)
        return 0

    lax.fori_loop(0, NCHUNK, chunk_body, 0)

    plsc.subcore_barrier()
    pltpu.sync_copy(out_sp.at[pl.ds(sid * ROWS_PT, ROWS_PT)],
                    out_hbm.at[cid, pl.ds(sid * ROWS_PT, ROWS_PT)])


@functools.partial(
    pl.kernel,
    out_type=(
        jax.ShapeDtypeStruct((EPAD, 16), jnp.float32),
        jax.ShapeDtypeStruct((2, NPAD, HID), jnp.float32),
    ),
    mesh=plsc.VectorSubcoreMesh(core_axis_name="c", subcore_axis_name="s"),
    compiler_params=_SC_PARAMS,
    scratch_types=[
        pltpu.VMEM((CHUNK,), jnp.int32),
        pltpu.VMEM((CHUNK,), jnp.int32),
        pltpu.VMEM((CHUNK, 16), jnp.float32),
        pltpu.VMEM((CHUNK, 16), jnp.float32),
        pltpu.VMEM((CHUNK, HID), jnp.float32),
        pltpu.SemaphoreType.DMA,
        pltpu.SemaphoreType.DMA,
        pltpu.VMEM_SHARED((NPAD, HID), jnp.float32),
    ],
)
def _sc_c(s_hbm, d_hbm, ex_hbm, dinv_hbm, h_hbm, att_hbm, out_hbm,
          sbuf, dbuf, exbuf, drows, hrows, sem, sem2, out_sp):
    _scc_body(s_hbm, d_hbm, ex_hbm, dinv_hbm, h_hbm, att_hbm, out_hbm,
              sbuf, dbuf, exbuf, drows, hrows, sem, sem2, out_sp)


_SEL = np.zeros((HID, HEADS), np.float32)
for _i in range(HID):
    _SEL[_i, _i // C_HEAD] = 1.0


def _mk_m(a_src, a_dst):
    return jnp.concatenate(
        [a_src.reshape(HID, 1) * _SEL, a_dst.reshape(HID, 1) * _SEL,
         jnp.zeros((HID, 6), jnp.float32)], axis=1)


def kernel(x, edge_index, batch, W1, a_src1, a_dst1, b1, W2, a_src2, a_dst2,
           b2, g1, be1, g2, be2, Wm, bm, Wr1, br1, gr, ber, Wr2, br2):
    loop = jnp.arange(N, dtype=edge_index.dtype)
    s = jnp.concatenate([edge_index[0], loop])
    d = jnp.concatenate([edge_index[1], loop])
    s = jnp.pad(s, (0, EPAD - E_TOT))
    d = jnp.pad(d, (0, EPAD - E_TOT))

    m1 = _mk_m(a_src1, a_dst1)
    m2 = _mk_m(a_src2, a_dst2)
    oh = (jnp.arange(G, dtype=batch.dtype)[:, None] == batch[None, :]
          ).astype(jnp.float32)

    h1r, asd1, c1 = _tc_pre(x, W1, m1)
    ex1, den1 = _sc_a(s, d, asd1, c1)
    dinv1 = _tc_dinv(den1)
    att1, agg1 = _sc_c(s, d, ex1, dinv1, h1r)

    h2r, asd2, c2 = _tc_mid(agg1, b1, g1, be1, W2, m2)
    ex2, den2 = _sc_a(s, d, asd2, c2)
    dinv2 = _tc_dinv(den2)
    att2, agg2 = _sc_c(s, d, ex2, dinv2, h2r)

    xo, rec = _tc_final(agg2, b2, g2, be2, Wm, bm, oh,
                        Wr1, br1, gr, ber, Wr2, br2)
    return (xo, rec, att1[:E_TOT, :HEADS], att2[:E_TOT, :HEADS])


# speedup vs baseline: 46.0093x; 1.1144x over previous
import functools

import numpy as np
import jax
import jax.numpy as jnp
from jax import lax
from jax.experimental import pallas as pl
from jax.experimental.pallas import tpu as pltpu
from jax.experimental.pallas import tpu_sc as plsc

N = 10000
E = 320000
HEADS = 5
C_HEAD = 32
HID = 160
G = 8
NC = 20

NWORK = 32
CHUNK_A = 96
CHUNK_C = 64
E_TOT = E + N
EPW = 10368
EPAD = NWORK * EPW
NCHUNK_A = EPW // CHUNK_A
NCHUNK_C = EPW // CHUNK_C
NPAD = 10240
ROWS_PT = NPAD // 16

_SC_PARAMS = pltpu.CompilerParams(
    needs_layout_passes=False, use_tc_tiling_on_sc=False)


def _ln(x, g, b):
    m = jnp.mean(x, axis=-1, keepdims=True)
    v = jnp.var(x, axis=-1, keepdims=True)
    return (x - m) / jnp.sqrt(v + 1e-5) * g + b


def _proj(h, m_ref, asd_ref, c_ref):
    asd = jnp.dot(h, m_ref[...], preferred_element_type=jnp.float32)
    asd_ref[...] = asd
    mx = jnp.max(asd[:, :2 * HEADS], axis=0)
    raw = mx[:HEADS] + mx[HEADS:]
    c = jnp.where(raw > 0, raw, 0.2 * raw)
    c_ref[...] = jnp.pad(c, (0, 11))


def _pre_body(x_ref, w_ref, m_ref, h_ref, asd_ref, c_ref):
    h = jnp.dot(x_ref[...], w_ref[...], preferred_element_type=jnp.float32)
    h_ref[...] = h
    _proj(h, m_ref, asd_ref, c_ref)


def _tc_pre(x, w, m):
    return pl.pallas_call(
        _pre_body,
        out_shape=(
            jax.ShapeDtypeStruct((N, HID), jnp.float32),
            jax.ShapeDtypeStruct((N, 16), jnp.float32),
            jax.ShapeDtypeStruct((16,), jnp.float32),
        ),
    )(x, w, m)


def _dinv_body(den_ref, o_ref):
    d = den_ref[0] + den_ref[1]
    o_ref[...] = 1.0 / jnp.maximum(d, 1e-30)


def _tc_dinv(den):
    return pl.pallas_call(
        _dinv_body,
        out_shape=jax.ShapeDtypeStruct((NPAD, 16), jnp.float32),
    )(den)


def _mid_body(agg_ref, b_ref, g_ref, be_ref, w_ref, m_ref,
              h2_ref, asd_ref, c_ref):
    o = agg_ref[0, :N] + agg_ref[1, :N] + b_ref[...]
    o = jnp.where(o > 0, o, 0.01 * o)
    h1 = _ln(o, g_ref[...], be_ref[...])
    h2 = jnp.dot(h1, w_ref[...], preferred_element_type=jnp.float32)
    h2_ref[...] = h2
    _proj(h2, m_ref, asd_ref, c_ref)


def _tc_mid(agg, b, g, be, w, m):
    return pl.pallas_call(
        _mid_body,
        out_shape=(
            jax.ShapeDtypeStruct((N, HID), jnp.float32),
            jax.ShapeDtypeStruct((N, 16), jnp.float32),
            jax.ShapeDtypeStruct((16,), jnp.float32),
        ),
    )(agg, b, g, be, w, m)


def _final_body(agg_ref, b_ref, g_ref, be_ref, wm_ref, bm_ref, oh_ref,
                wr1_ref, br1_ref, gr_ref, ber_ref, wr2_ref, br2_ref,
                xo_ref, rec_ref):
    o = agg_ref[0, :N] + agg_ref[1, :N] + b_ref[...]
    o = jnp.where(o > 0, o, 0.01 * o)
    h2 = _ln(o, g_ref[...], be_ref[...])
    xo = jnp.dot(h2, wm_ref[...], preferred_element_type=jnp.float32)
    xo = xo + bm_ref[...]
    xo = jnp.where(xo > 0, xo, 0.01 * xo)
    xo_ref[...] = xo
    oh = oh_ref[...]
    pooled = jnp.dot(oh, xo, preferred_element_type=jnp.float32)
    cnt = jnp.sum(oh, axis=1)
    pooled = pooled / jnp.maximum(cnt, 1.0)[:, None]
    r = jnp.dot(pooled, wr1_ref[...], preferred_element_type=jnp.float32)
    r = r + br1_ref[...]
    r = _ln(r, gr_ref[...], ber_ref[...])
    r = jnp.maximum(r, 0.0)
    rec = jnp.dot(r, wr2_ref[...], preferred_element_type=jnp.float32)
    rec_ref[...] = rec + br2_ref[...]


def _tc_final(agg, b, g, be, wm, bm, oh, wr1, br1, gr, ber, wr2, br2):
    return pl.pallas_call(
        _final_body,
        out_shape=(
            jax.ShapeDtypeStruct((N, HID), jnp.float32),
            jax.ShapeDtypeStruct((G, NC), jnp.float32),
        ),
    )(agg, b, g, be, wm, bm, oh, wr1, br1, gr, ber, wr2, br2)


def _sca_body(s_hbm, d_hbm, asd_hbm, c_hbm, ex_hbm, den_hbm,
              cbuf, sbufs, dbufs, exbufs, arowss, browss, sems, den_sp):
  cid = lax.axis_index("c")
  sid = lax.axis_index("s")
  wid = sid * 2 + cid
  base = wid * EPW

  pltpu.sync_copy(c_hbm, cbuf)
  cvec = cbuf[...]

  for b in range(2):
    def zex(i, _, b=b):
      exbufs[b][i, :] = jnp.zeros((16,), jnp.float32)
      return 0
    lax.fori_loop(0, CHUNK_A, zex, 0)
  nz = ROWS_PT // CHUNK_A + 1
  for b5 in range(nz):
    r0 = b5 * CHUNK_A
    rows = min(CHUNK_A, ROWS_PT - r0)
    pltpu.sync_copy(exbufs[0].at[pl.ds(0, rows)],
                    den_sp.at[pl.ds(sid * ROWS_PT + r0, rows)])
  plsc.subcore_barrier()

  iota = lax.broadcasted_iota(jnp.int32, (16,), 0)

  def load_idx(ci, b):
    off = base + ci * CHUNK_A
    pltpu.sync_copy(s_hbm.at[pl.ds(off, CHUNK_A)], sbufs[b])
    pltpu.sync_copy(d_hbm.at[pl.ds(off, CHUNK_A)], dbufs[b])

  def issue(b):
    pltpu.async_copy(asd_hbm.at[sbufs[b]], arowss[b], sems[2 * b])
    pltpu.async_copy(asd_hbm.at[dbufs[b]], browss[b], sems[2 * b + 1])

  def compute(ci, b):
    off = base + ci * CHUNK_A
    pltpu.make_async_copy(asd_hbm.at[sbufs[b]], arowss[b], sems[2 * b]).wait()
    pltpu.make_async_copy(asd_hbm.at[dbufs[b]], browss[b],
                          sems[2 * b + 1]).wait()
    arows, brows, exbuf = arowss[b], browss[b], exbufs[b]
    for j in range(CHUNK_A // 16):
      eid = off + j * 16 + iota
      valid = eid < E_TOT
      row = jnp.full((16,), j * 16, jnp.int32) + iota
      for hd in range(HEADS):
        hvec = jnp.full((16,), hd, jnp.int32)
        a = plsc.load_gather(arows, [row, hvec])
        bb = plsc.load_gather(brows, [row, hvec + HEADS])
        al = a + bb
        al = jnp.where(al > 0, al, 0.2 * al)
        ex = jnp.exp(al - cvec[hd])
        ex = jnp.where(valid, ex, 0.0)
        plsc.store_scatter(exbuf, [row, hvec], ex)
    pltpu.sync_copy(exbuf, ex_hbm.at[pl.ds(off, CHUNK_A)])
    pltpu.sync_copy(exbuf, den_sp.at[dbufs[b]], add=True)

  load_idx(0, 0)
  issue(0)

  def pair(g, _):
    i0 = 2 * g
    load_idx(i0 + 1, 1)
    issue(1)
    compute(i0, 0)
    load_idx(i0 + 2, 0)
    issue(0)
    compute(i0 + 1, 1)
    return 0

  lax.fori_loop(0, NCHUNK_A // 2 - 1, pair, 0)
  i0 = NCHUNK_A - 2
  load_idx(i0 + 1, 1)
  issue(1)
  compute(i0, 0)
  compute(i0 + 1, 1)

  plsc.subcore_barrier()
  pltpu.sync_copy(den_sp.at[pl.ds(sid * ROWS_PT, ROWS_PT)],
                  den_hbm.at[cid, pl.ds(sid * ROWS_PT, ROWS_PT)])


@functools.partial(
    pl.kernel,
    out_type=(
        jax.ShapeDtypeStruct((EPAD, 16), jnp.float32),
        jax.ShapeDtypeStruct((2, NPAD, 16), jnp.float32),
    ),
    mesh=plsc.VectorSubcoreMesh(core_axis_name="c", subcore_axis_name="s"),
    compiler_params=_SC_PARAMS,
    scratch_types=[
        pltpu.VMEM((16,), jnp.float32),
        [pltpu.VMEM((CHUNK_A,), jnp.int32)] * 2,
        [pltpu.VMEM((CHUNK_A,), jnp.int32)] * 2,
        [pltpu.VMEM((CHUNK_A, 16), jnp.float32)] * 2,
        [pltpu.VMEM((CHUNK_A, 16), jnp.float32)] * 2,
        [pltpu.VMEM((CHUNK_A, 16), jnp.float32)] * 2,
        [pltpu.SemaphoreType.DMA] * 4,
        pltpu.VMEM_SHARED((NPAD, 16), jnp.float32),
    ],
)
def _sc_a(s_hbm, d_hbm, asd_hbm, c_hbm, ex_hbm, den_hbm,
          cbuf, sbufs, dbufs, exbufs, arowss, browss, sems, den_sp):
    _sca_body(s_hbm, d_hbm, asd_hbm, c_hbm, ex_hbm, den_hbm,
              cbuf, sbufs, dbufs, exbufs, arowss, browss, sems, den_sp)


def _scc_body(s_hbm, d_hbm, ex_hbm, dinv_hbm, h_hbm, att_hbm, out_hbm,
              sbufs, dbufs, exbufs, drowss, hrowss, sems, out_sp):
  cid = lax.axis_index("c")
  sid = lax.axis_index("s")
  wid = sid * 2 + cid
  base = wid * EPW

  def zrow(i, _):
    for k in range(HID // 16):
      hrowss[0][i, pl.ds(k * 16, 16)] = jnp.zeros((16,), jnp.float32)
    return 0
  lax.fori_loop(0, CHUNK_C, zrow, 0)
  for b5 in range(ROWS_PT // CHUNK_C):
    pltpu.sync_copy(hrowss[0],
                    out_sp.at[pl.ds(sid * ROWS_PT + b5 * CHUNK_C, CHUNK_C)])
  plsc.subcore_barrier()

  def load_idx(ci, b):
    off = base + ci * CHUNK_C
    pltpu.sync_copy(s_hbm.at[pl.ds(off, CHUNK_C)], sbufs[b])
    pltpu.sync_copy(d_hbm.at[pl.ds(off, CHUNK_C)], dbufs[b])
    pltpu.sync_copy(ex_hbm.at[pl.ds(off, CHUNK_C)], exbufs[b])

  def issue(b):
    pltpu.async_copy(h_hbm.at[sbufs[b]], hrowss[b], sems[2 * b])
    pltpu.async_copy(dinv_hbm.at[dbufs[b]], drowss[b], sems[2 * b + 1])

  def compute(ci, b):
    off = base + ci * CHUNK_C
    pltpu.make_async_copy(h_hbm.at[sbufs[b]], hrowss[b], sems[2 * b]).wait()
    pltpu.make_async_copy(dinv_hbm.at[dbufs[b]], drowss[b],
                          sems[2 * b + 1]).wait()
    exbuf, drows, hrows = exbufs[b], drowss[b], hrowss[b]

    def scale(e, _):
      av = exbuf[e] * drows[e]
      exbuf[e] = av
      for k in range(HID // 16):
        sl = pl.ds(k * 16, 16)
        hrows[e, sl] = hrows[e, sl] * av[k // 2]
      return 0
    lax.fori_loop(0, CHUNK_C, scale, 0)

    pltpu.sync_copy(exbuf, att_hbm.at[pl.ds(off, CHUNK_C)])
    pltpu.sync_copy(hrows, out_sp.at[dbufs[b]], add=True)

  load_idx(0, 0)
  issue(0)

  def pair(g, _):
    i0 = 2 * g
    load_idx(i0 + 1, 1)
    issue(1)
    compute(i0, 0)
    load_idx(i0 + 2, 0)
    issue(0)
    compute(i0 + 1, 1)
    return 0

  lax.fori_loop(0, NCHUNK_C // 2 - 1, pair, 0)
  i0 = NCHUNK_C - 2
  load_idx(i0 + 1, 1)
  issue(1)
  compute(i0, 0)
  compute(i0 + 1, 1)

  plsc.subcore_barrier()
  pltpu.sync_copy(out_sp.at[pl.ds(sid * ROWS_PT, ROWS_PT)],
                  out_hbm.at[cid, pl.ds(sid * ROWS_PT, ROWS_PT)])


@functools.partial(
    pl.kernel,
    out_type=(
        jax.ShapeDtypeStruct((EPAD, 16), jnp.float32),
        jax.ShapeDtypeStruct((2, NPAD, HID), jnp.float32),
    ),
    mesh=plsc.VectorSubcoreMesh(core_axis_name="c", subcore_axis_name="s"),
    compiler_params=_SC_PARAMS,
    scratch_types=[
        [pltpu.VMEM((CHUNK_C,), jnp.int32)] * 2,
        [pltpu.VMEM((CHUNK_C,), jnp.int32)] * 2,
        [pltpu.VMEM((CHUNK_C, 16), jnp.float32)] * 2,
        [pltpu.VMEM((CHUNK_C, 16), jnp.float32)] * 2,
        [pltpu.VMEM((CHUNK_C, HID), jnp.float32)] * 2,
        [pltpu.SemaphoreType.DMA] * 4,
        pltpu.VMEM_SHARED((NPAD, HID), jnp.float32),
    ],
)
def _sc_c(s_hbm, d_hbm, ex_hbm, dinv_hbm, h_hbm, att_hbm, out_hbm,
          sbufs, dbufs, exbufs, drowss, hrowss, sems, out_sp):
    _scc_body(s_hbm, d_hbm, ex_hbm, dinv_hbm, h_hbm, att_hbm, out_hbm,
              sbufs, dbufs, exbufs, drowss, hrowss, sems, out_sp)


_SEL = np.zeros((HID, HEADS), np.float32)
for _i in range(HID):
    _SEL[_i, _i // C_HEAD] = 1.0


def _mk_m(a_src, a_dst):
    return jnp.concatenate(
        [a_src.reshape(HID, 1) * _SEL, a_dst.reshape(HID, 1) * _SEL,
         jnp.zeros((HID, 6), jnp.float32)], axis=1)


def kernel(x, edge_index, batch, W1, a_src1, a_dst1, b1, W2, a_src2, a_dst2,
           b2, g1, be1, g2, be2, Wm, bm, Wr1, br1, gr, ber, Wr2, br2):
    loop = jnp.arange(N, dtype=edge_index.dtype)
    s = jnp.concatenate([edge_index[0], loop])
    d = jnp.concatenate([edge_index[1], loop])
    s = jnp.pad(s, (0, EPAD - E_TOT))
    d = jnp.pad(d, (0, EPAD - E_TOT))

    m1 = _mk_m(a_src1, a_dst1)
    m2 = _mk_m(a_src2, a_dst2)
    oh = (jnp.arange(G, dtype=batch.dtype)[:, None] == batch[None, :]
          ).astype(jnp.float32)

    h1r, asd1, c1 = _tc_pre(x, W1, m1)
    ex1, den1 = _sc_a(s, d, asd1, c1)
    dinv1 = _tc_dinv(den1)
    att1, agg1 = _sc_c(s, d, ex1, dinv1, h1r)

    h2r, asd2, c2 = _tc_mid(agg1, b1, g1, be1, W2, m2)
    ex2, den2 = _sc_a(s, d, asd2, c2)
    dinv2 = _tc_dinv(den2)
    att2, agg2 = _sc_c(s, d, ex2, dinv2, h2r)

    xo, rec = _tc_final(agg2, b2, g2, be2, Wm, bm, oh,
                        Wr1, br1, gr, ber, Wr2, br2)
    return (xo, rec, att1[:E_TOT, :HEADS], att2[:E_TOT, :HEADS])


# speedup vs baseline: 46.5951x; 1.0127x over previous
import functools

import numpy as np
import jax
import jax.numpy as jnp
from jax import lax
from jax.experimental import pallas as pl
from jax.experimental.pallas import tpu as pltpu
from jax.experimental.pallas import tpu_sc as plsc

N = 10000
E = 320000
HEADS = 5
C_HEAD = 32
HID = 160
G = 8
NC = 20

NWORK = 32
CHUNK_A = 96
CHUNK_C = 64
E_TOT = E + N
EPW = 10368
EPAD = NWORK * EPW
NCHUNK_A = EPW // CHUNK_A
NCHUNK_C = EPW // CHUNK_C
NPAD = 10240
ROWS_PT = NPAD // 16

_SC_PARAMS = pltpu.CompilerParams(
    needs_layout_passes=False, use_tc_tiling_on_sc=False)


def _ln(x, g, b):
    m = jnp.mean(x, axis=-1, keepdims=True)
    v = jnp.var(x, axis=-1, keepdims=True)
    return (x - m) / jnp.sqrt(v + 1e-5) * g + b


def _proj(h, m_ref, asd_ref, c_ref):
    asd = jnp.dot(h, m_ref[...], preferred_element_type=jnp.float32)
    asd_ref[...] = asd
    mx = jnp.max(asd[:, :2 * HEADS], axis=0)
    raw = mx[:HEADS] + mx[HEADS:]
    c = jnp.where(raw > 0, raw, 0.2 * raw)
    c_ref[...] = jnp.pad(c, (0, 11))


def _pre_body(x_ref, w_ref, m_ref, h_ref, asd_ref, c_ref):
    h = jnp.dot(x_ref[...], w_ref[...], preferred_element_type=jnp.float32)
    h_ref[...] = h
    _proj(h, m_ref, asd_ref, c_ref)


def _tc_pre(x, w, m):
    return pl.pallas_call(
        _pre_body,
        out_shape=(
            jax.ShapeDtypeStruct((N, HID), jnp.float32),
            jax.ShapeDtypeStruct((N, 16), jnp.float32),
            jax.ShapeDtypeStruct((16,), jnp.float32),
        ),
    )(x, w, m)


def _dinv_body(den_ref, o_ref):
    d = den_ref[0] + den_ref[1]
    o_ref[...] = 1.0 / jnp.maximum(d, 1e-30)


def _tc_dinv(den):
    return pl.pallas_call(
        _dinv_body,
        out_shape=jax.ShapeDtypeStruct((NPAD, 16), jnp.float32),
    )(den)


def _mid_body(agg_ref, b_ref, g_ref, be_ref, w_ref, m_ref,
              h2_ref, asd_ref, c_ref):
    o = agg_ref[0, :N] + agg_ref[1, :N] + b_ref[...]
    o = jnp.where(o > 0, o, 0.01 * o)
    h1 = _ln(o, g_ref[...], be_ref[...])
    h2 = jnp.dot(h1, w_ref[...], preferred_element_type=jnp.float32)
    h2_ref[...] = h2
    _proj(h2, m_ref, asd_ref, c_ref)


def _tc_mid(agg, b, g, be, w, m):
    return pl.pallas_call(
        _mid_body,
        out_shape=(
            jax.ShapeDtypeStruct((N, HID), jnp.float32),
            jax.ShapeDtypeStruct((N, 16), jnp.float32),
            jax.ShapeDtypeStruct((16,), jnp.float32),
        ),
    )(agg, b, g, be, w, m)


def _final_body(agg_ref, b_ref, g_ref, be_ref, wm_ref, bm_ref, oh_ref,
                wr1_ref, br1_ref, gr_ref, ber_ref, wr2_ref, br2_ref,
                xo_ref, rec_ref):
    o = agg_ref[0, :N] + agg_ref[1, :N] + b_ref[...]
    o = jnp.where(o > 0, o, 0.01 * o)
    h2 = _ln(o, g_ref[...], be_ref[...])
    xo = jnp.dot(h2, wm_ref[...], preferred_element_type=jnp.float32)
    xo = xo + bm_ref[...]
    xo = jnp.where(xo > 0, xo, 0.01 * xo)
    xo_ref[...] = xo
    oh = oh_ref[...]
    pooled = jnp.dot(oh, xo, preferred_element_type=jnp.float32)
    cnt = jnp.sum(oh, axis=1)
    pooled = pooled / jnp.maximum(cnt, 1.0)[:, None]
    r = jnp.dot(pooled, wr1_ref[...], preferred_element_type=jnp.float32)
    r = r + br1_ref[...]
    r = _ln(r, gr_ref[...], ber_ref[...])
    r = jnp.maximum(r, 0.0)
    rec = jnp.dot(r, wr2_ref[...], preferred_element_type=jnp.float32)
    rec_ref[...] = rec + br2_ref[...]


def _tc_final(agg, b, g, be, wm, bm, oh, wr1, br1, gr, ber, wr2, br2):
    return pl.pallas_call(
        _final_body,
        out_shape=(
            jax.ShapeDtypeStruct((N, HID), jnp.float32),
            jax.ShapeDtypeStruct((G, NC), jnp.float32),
        ),
    )(agg, b, g, be, wm, bm, oh, wr1, br1, gr, ber, wr2, br2)


def _sca_body(s_hbm, d_hbm, asd_hbm, c_hbm, ex_hbm, den_hbm,
              cbuf, sbufs, dbufs, exbufs, arowss, browss, sems, den_sp):
  cid = lax.axis_index("c")
  sid = lax.axis_index("s")
  wid = sid * 2 + cid
  base = wid * EPW

  pltpu.sync_copy(c_hbm, cbuf)
  cvec = cbuf[...]

  for b in range(2):
    def zex(i, _, b=b):
      exbufs[b][i, :] = jnp.zeros((16,), jnp.float32)
      return 0
    lax.fori_loop(0, CHUNK_A, zex, 0)
  nz = ROWS_PT // CHUNK_A + 1
  for b5 in range(nz):
    r0 = b5 * CHUNK_A
    rows = min(CHUNK_A, ROWS_PT - r0)
    pltpu.sync_copy(exbufs[0].at[pl.ds(0, rows)],
                    den_sp.at[pl.ds(sid * ROWS_PT + r0, rows)])
  plsc.subcore_barrier()

  iota = lax.broadcasted_iota(jnp.int32, (16,), 0)

  def load_idx(ci, b):
    off = base + ci * CHUNK_A
    pltpu.sync_copy(s_hbm.at[pl.ds(off, CHUNK_A)], sbufs[b])
    pltpu.sync_copy(d_hbm.at[pl.ds(off, CHUNK_A)], dbufs[b])

  def issue(b):
    pltpu.async_copy(asd_hbm.at[sbufs[b]], arowss[b], sems[2 * b])
    pltpu.async_copy(asd_hbm.at[dbufs[b]], browss[b], sems[2 * b + 1])

  def compute(ci, b, first):
    off = base + ci * CHUNK_A
    pltpu.make_async_copy(asd_hbm.at[sbufs[b]], arowss[b], sems[2 * b]).wait()
    pltpu.make_async_copy(asd_hbm.at[dbufs[b]], browss[b],
                          sems[2 * b + 1]).wait()
    arows, brows, exbuf = arowss[b], browss[b], exbufs[b]
    if not first:
      pltpu.make_async_copy(exbuf, ex_hbm.at[pl.ds(off, CHUNK_A)],
                            sems[4 + 2 * b]).wait()
    for j in range(CHUNK_A // 16):
      eid = off + j * 16 + iota
      valid = eid < E_TOT
      row = jnp.full((16,), j * 16, jnp.int32) + iota
      for hd in range(HEADS):
        hvec = jnp.full((16,), hd, jnp.int32)
        a = plsc.load_gather(arows, [row, hvec])
        bb = plsc.load_gather(brows, [row, hvec + HEADS])
        al = a + bb
        al = jnp.where(al > 0, al, 0.2 * al)
        ex = jnp.exp(al - cvec[hd])
        ex = jnp.where(valid, ex, 0.0)
        plsc.store_scatter(exbuf, [row, hvec], ex)
    pltpu.async_copy(exbuf, ex_hbm.at[pl.ds(off, CHUNK_A)], sems[4 + 2 * b])
    pltpu.sync_copy(exbuf, den_sp.at[dbufs[b]], add=True)

  load_idx(0, 0)
  issue(0)
  load_idx(1, 1)
  issue(1)
  compute(0, 0, True)
  load_idx(2, 0)
  issue(0)
  compute(1, 1, True)

  def pair(g, _):
    i0 = 2 * g
    load_idx(i0 + 1, 1)
    issue(1)
    compute(i0, 0, False)
    load_idx(i0 + 2, 0)
    issue(0)
    compute(i0 + 1, 1, False)
    return 0

  lax.fori_loop(1, NCHUNK_A // 2 - 1, pair, 0)
  i0 = NCHUNK_A - 2
  load_idx(i0 + 1, 1)
  issue(1)
  compute(i0, 0, False)
  compute(i0 + 1, 1, False)
  for b in range(2):
    pltpu.make_async_copy(exbufs[b], ex_hbm.at[pl.ds(base, CHUNK_A)],
                          sems[4 + 2 * b]).wait()

  plsc.subcore_barrier()
  pltpu.sync_copy(den_sp.at[pl.ds(sid * ROWS_PT, ROWS_PT)],
                  den_hbm.at[cid, pl.ds(sid * ROWS_PT, ROWS_PT)])


@functools.partial(
    pl.kernel,
    out_type=(
        jax.ShapeDtypeStruct((EPAD, 16), jnp.float32),
        jax.ShapeDtypeStruct((2, NPAD, 16), jnp.float32),
    ),
    mesh=plsc.VectorSubcoreMesh(core_axis_name="c", subcore_axis_name="s"),
    compiler_params=_SC_PARAMS,
    scratch_types=[
        pltpu.VMEM((16,), jnp.float32),
        [pltpu.VMEM((CHUNK_A,), jnp.int32)] * 2,
        [pltpu.VMEM((CHUNK_A,), jnp.int32)] * 2,
        [pltpu.VMEM((CHUNK_A, 16), jnp.float32)] * 2,
        [pltpu.VMEM((CHUNK_A, 16), jnp.float32)] * 2,
        [pltpu.VMEM((CHUNK_A, 16), jnp.float32)] * 2,
        [pltpu.SemaphoreType.DMA] * 8,
        pltpu.VMEM_SHARED((NPAD, 16), jnp.float32),
    ],
)
def _sc_a(s_hbm, d_hbm, asd_hbm, c_hbm, ex_hbm, den_hbm,
          cbuf, sbufs, dbufs, exbufs, arowss, browss, sems, den_sp):
    _sca_body(s_hbm, d_hbm, asd_hbm, c_hbm, ex_hbm, den_hbm,
              cbuf, sbufs, dbufs, exbufs, arowss, browss, sems, den_sp)


def _scc_body(s_hbm, d_hbm, ex_hbm, dinv_hbm, h_hbm, att_hbm, out_hbm,
              sbufs, dbufs, exbufs, drowss, hrowss, sems, out_sp):
  cid = lax.axis_index("c")
  sid = lax.axis_index("s")
  wid = sid * 2 + cid
  base = wid * EPW

  def zrow(i, _):
    for k in range(HID // 16):
      hrowss[0][i, pl.ds(k * 16, 16)] = jnp.zeros((16,), jnp.float32)
    return 0
  lax.fori_loop(0, CHUNK_C, zrow, 0)
  for b5 in range(ROWS_PT // CHUNK_C):
    pltpu.sync_copy(hrowss[0],
                    out_sp.at[pl.ds(sid * ROWS_PT + b5 * CHUNK_C, CHUNK_C)])
  plsc.subcore_barrier()

  def load_idx(ci, b):
    off = base + ci * CHUNK_C
    pltpu.sync_copy(s_hbm.at[pl.ds(off, CHUNK_C)], sbufs[b])
    pltpu.sync_copy(d_hbm.at[pl.ds(off, CHUNK_C)], dbufs[b])

  def issue(b):
    pltpu.async_copy(h_hbm.at[sbufs[b]], hrowss[b], sems[2 * b])
    pltpu.async_copy(dinv_hbm.at[dbufs[b]], drowss[b], sems[2 * b + 1])

  def compute(ci, b, first):
    off = base + ci * CHUNK_C
    exbuf, drows, hrows = exbufs[b], drowss[b], hrowss[b]
    if not first:
      pltpu.make_async_copy(exbuf, att_hbm.at[pl.ds(off, CHUNK_C)],
                            sems[4 + 2 * b]).wait()
    pltpu.sync_copy(ex_hbm.at[pl.ds(off, CHUNK_C)], exbuf)
    pltpu.make_async_copy(h_hbm.at[sbufs[b]], hrowss[b], sems[2 * b]).wait()
    pltpu.make_async_copy(dinv_hbm.at[dbufs[b]], drowss[b],
                          sems[2 * b + 1]).wait()

    def scale(e, _):
      av = exbuf[e] * drows[e]
      exbuf[e] = av
      for k in range(HID // 16):
        sl = pl.ds(k * 16, 16)
        hrows[e, sl] = hrows[e, sl] * av[k // 2]
      return 0
    lax.fori_loop(0, CHUNK_C, scale, 0)

    pltpu.async_copy(exbuf, att_hbm.at[pl.ds(off, CHUNK_C)], sems[4 + 2 * b])
    pltpu.sync_copy(hrows, out_sp.at[dbufs[b]], add=True)

  load_idx(0, 0)
  issue(0)
  load_idx(1, 1)
  issue(1)
  compute(0, 0, True)
  load_idx(2, 0)
  issue(0)
  compute(1, 1, True)

  def pair(g, _):
    i0 = 2 * g
    load_idx(i0 + 1, 1)
    issue(1)
    compute(i0, 0, False)
    load_idx(i0 + 2, 0)
    issue(0)
    compute(i0 + 1, 1, False)
    return 0

  lax.fori_loop(1, NCHUNK_C // 2 - 1, pair, 0)
  i0 = NCHUNK_C - 2
  load_idx(i0 + 1, 1)
  issue(1)
  compute(i0, 0, False)
  compute(i0 + 1, 1, False)
  for b in range(2):
    pltpu.make_async_copy(exbufs[b], att_hbm.at[pl.ds(base, CHUNK_C)],
                          sems[4 + 2 * b]).wait()

  plsc.subcore_barrier()
  pltpu.sync_copy(out_sp.at[pl.ds(sid * ROWS_PT, ROWS_PT)],
                  out_hbm.at[cid, pl.ds(sid * ROWS_PT, ROWS_PT)])


@functools.partial(
    pl.kernel,
    out_type=(
        jax.ShapeDtypeStruct((EPAD, 16), jnp.float32),
        jax.ShapeDtypeStruct((2, NPAD, HID), jnp.float32),
    ),
    mesh=plsc.VectorSubcoreMesh(core_axis_name="c", subcore_axis_name="s"),
    compiler_params=_SC_PARAMS,
    scratch_types=[
        [pltpu.VMEM((CHUNK_C,), jnp.int32)] * 2,
        [pltpu.VMEM((CHUNK_C,), jnp.int32)] * 2,
        [pltpu.VMEM((CHUNK_C, 16), jnp.float32)] * 2,
        [pltpu.VMEM((CHUNK_C, 16), jnp.float32)] * 2,
        [pltpu.VMEM((CHUNK_C, HID), jnp.float32)] * 2,
        [pltpu.SemaphoreType.DMA] * 8,
        pltpu.VMEM_SHARED((NPAD, HID), jnp.float32),
    ],
)
def _sc_c(s_hbm, d_hbm, ex_hbm, dinv_hbm, h_hbm, att_hbm, out_hbm,
          sbufs, dbufs, exbufs, drowss, hrowss, sems, out_sp):
    _scc_body(s_hbm, d_hbm, ex_hbm, dinv_hbm, h_hbm, att_hbm, out_hbm,
              sbufs, dbufs, exbufs, drowss, hrowss, sems, out_sp)


_SEL = np.zeros((HID, HEADS), np.float32)
for _i in range(HID):
    _SEL[_i, _i // C_HEAD] = 1.0


def _mk_m(a_src, a_dst):
    return jnp.concatenate(
        [a_src.reshape(HID, 1) * _SEL, a_dst.reshape(HID, 1) * _SEL,
         jnp.zeros((HID, 6), jnp.float32)], axis=1)


def kernel(x, edge_index, batch, W1, a_src1, a_dst1, b1, W2, a_src2, a_dst2,
           b2, g1, be1, g2, be2, Wm, bm, Wr1, br1, gr, ber, Wr2, br2):
    loop = jnp.arange(N, dtype=edge_index.dtype)
    s = jnp.concatenate([edge_index[0], loop])
    d = jnp.concatenate([edge_index[1], loop])
    s = jnp.pad(s, (0, EPAD - E_TOT))
    d = jnp.pad(d, (0, EPAD - E_TOT))

    m1 = _mk_m(a_src1, a_dst1)
    m2 = _mk_m(a_src2, a_dst2)
    oh = (jnp.arange(G, dtype=batch.dtype)[:, None] == batch[None, :]
          ).astype(jnp.float32)

    h1r, asd1, c1 = _tc_pre(x, W1, m1)
    ex1, den1 = _sc_a(s, d, asd1, c1)
    dinv1 = _tc_dinv(den1)
    att1, agg1 = _sc_c(s, d, ex1, dinv1, h1r)

    h2r, asd2, c2 = _tc_mid(agg1, b1, g1, be1, W2, m2)
    ex2, den2 = _sc_a(s, d, asd2, c2)
    dinv2 = _tc_dinv(den2)
    att2, agg2 = _sc_c(s, d, ex2, dinv2, h2r)

    xo, rec = _tc_final(agg2, b2, g2, be2, Wm, bm, oh,
                        Wr1, br1, gr, ber, Wr2, br2)
    return (xo, rec, att1[:E_TOT, :HEADS], att2[:E_TOT, :HEADS])


# speedup vs baseline: 50.5682x; 1.0853x over previous
import functools

import numpy as np
import jax
import jax.numpy as jnp
from jax import lax
from jax.experimental import pallas as pl
from jax.experimental.pallas import tpu as pltpu
from jax.experimental.pallas import tpu_sc as plsc

N = 10000
E = 320000
HEADS = 5
C_HEAD = 32
HID = 160
G = 8
NC = 20

NWORK = 32
CHUNK_A = 96
CHUNK_C = 64
E_TOT = E + N
EPW = 10368
EPAD = NWORK * EPW
NCHUNK_A = EPW // CHUNK_A
NCHUNK_C = EPW // CHUNK_C
NPAD = 10240
ROWS_PT = NPAD // 16

_SC_PARAMS = pltpu.CompilerParams(
    needs_layout_passes=False, use_tc_tiling_on_sc=False)


def _ln(x, g, b):
    m = jnp.mean(x, axis=-1, keepdims=True)
    v = jnp.var(x, axis=-1, keepdims=True)
    return (x - m) / jnp.sqrt(v + 1e-5) * g + b


def _proj(h, m_ref, asd_ref, c_ref):
    asd = jnp.dot(h, m_ref[...], preferred_element_type=jnp.float32)
    asd_ref[...] = asd
    mx = jnp.max(asd[:, :2 * HEADS], axis=0)
    raw = mx[:HEADS] + mx[HEADS:]
    c = jnp.where(raw > 0, raw, 0.2 * raw)
    c_ref[...] = jnp.pad(c, (0, 11))


def _pre_body(x_ref, w_ref, m_ref, h_ref, asd_ref, c_ref):
    h = jnp.dot(x_ref[...], w_ref[...], preferred_element_type=jnp.float32)
    h_ref[...] = h
    _proj(h, m_ref, asd_ref, c_ref)


def _tc_pre(x, w, m):
    return pl.pallas_call(
        _pre_body,
        out_shape=(
            jax.ShapeDtypeStruct((N, HID), jnp.float32),
            jax.ShapeDtypeStruct((N, 16), jnp.float32),
            jax.ShapeDtypeStruct((16,), jnp.float32),
        ),
    )(x, w, m)


def _dinv_body(den_ref, o_ref):
    d = den_ref[0] + den_ref[1]
    o_ref[...] = 1.0 / jnp.maximum(d, 1e-30)


def _tc_dinv(den):
    return pl.pallas_call(
        _dinv_body,
        out_shape=jax.ShapeDtypeStruct((NPAD, 16), jnp.float32),
    )(den)


def _mid_body(agg_ref, b_ref, g_ref, be_ref, w_ref, m_ref,
              h2_ref, asd_ref, c_ref):
    o = agg_ref[0, :N] + agg_ref[1, :N] + b_ref[...]
    o = jnp.where(o > 0, o, 0.01 * o)
    h1 = _ln(o, g_ref[...], be_ref[...])
    h2 = jnp.dot(h1, w_ref[...], preferred_element_type=jnp.float32)
    h2_ref[...] = h2
    _proj(h2, m_ref, asd_ref, c_ref)


def _tc_mid(agg, b, g, be, w, m):
    return pl.pallas_call(
        _mid_body,
        out_shape=(
            jax.ShapeDtypeStruct((N, HID), jnp.float32),
            jax.ShapeDtypeStruct((N, 16), jnp.float32),
            jax.ShapeDtypeStruct((16,), jnp.float32),
        ),
    )(agg, b, g, be, w, m)


def _final_body(agg_ref, b_ref, g_ref, be_ref, wm_ref, bm_ref, oh_ref,
                wr1_ref, br1_ref, gr_ref, ber_ref, wr2_ref, br2_ref,
                xo_ref, rec_ref):
    o = agg_ref[0, :N] + agg_ref[1, :N] + b_ref[...]
    o = jnp.where(o > 0, o, 0.01 * o)
    h2 = _ln(o, g_ref[...], be_ref[...])
    xo = jnp.dot(h2, wm_ref[...], preferred_element_type=jnp.float32)
    xo = xo + bm_ref[...]
    xo = jnp.where(xo > 0, xo, 0.01 * xo)
    xo_ref[...] = xo
    oh = oh_ref[...]
    pooled = jnp.dot(oh, xo, preferred_element_type=jnp.float32)
    cnt = jnp.sum(oh, axis=1)
    pooled = pooled / jnp.maximum(cnt, 1.0)[:, None]
    r = jnp.dot(pooled, wr1_ref[...], preferred_element_type=jnp.float32)
    r = r + br1_ref[...]
    r = _ln(r, gr_ref[...], ber_ref[...])
    r = jnp.maximum(r, 0.0)
    rec = jnp.dot(r, wr2_ref[...], preferred_element_type=jnp.float32)
    rec_ref[...] = rec + br2_ref[...]


def _tc_final(agg, b, g, be, wm, bm, oh, wr1, br1, gr, ber, wr2, br2):
    return pl.pallas_call(
        _final_body,
        out_shape=(
            jax.ShapeDtypeStruct((N, HID), jnp.float32),
            jax.ShapeDtypeStruct((G, NC), jnp.float32),
        ),
    )(agg, b, g, be, wm, bm, oh, wr1, br1, gr, ber, wr2, br2)


def _sca_body(s_hbm, d_hbm, asd_hbm, c_hbm, ex_hbm, den_hbm,
              cbuf, sbig, dbig, dadds, exbufs, arowss, browss, sems, den_sp):
  cid = lax.axis_index("c")
  sid = lax.axis_index("s")
  wid = sid * 2 + cid
  base = wid * EPW

  pltpu.sync_copy(c_hbm, cbuf)
  cvec = cbuf[...]
  pltpu.sync_copy(s_hbm.at[pl.ds(base, EPW)], sbig)
  pltpu.sync_copy(d_hbm.at[pl.ds(base, EPW)], dbig)

  for b in range(2):
    def zex(i, _, b=b):
      exbufs[b][i, :] = jnp.zeros((16,), jnp.float32)
      return 0
    lax.fori_loop(0, CHUNK_A, zex, 0)
  nz = ROWS_PT // CHUNK_A + 1
  for b5 in range(nz):
    r0 = b5 * CHUNK_A
    rows = min(CHUNK_A, ROWS_PT - r0)
    pltpu.sync_copy(exbufs[0].at[pl.ds(0, rows)],
                    den_sp.at[pl.ds(sid * ROWS_PT + r0, rows)])
  plsc.subcore_barrier()

  iota = lax.broadcasted_iota(jnp.int32, (16,), 0)

  def load_idx(ci, b):
    del ci, b

  def issue(b, ci):
    loc = ci * CHUNK_A
    pltpu.async_copy(asd_hbm.at[sbig.at[pl.ds(loc, CHUNK_A)]],
                     arowss[b], sems[2 * b])
    pltpu.async_copy(asd_hbm.at[dbig.at[pl.ds(loc, CHUNK_A)]],
                     browss[b], sems[2 * b + 1])

  def compute(ci, b, first):
    off = base + ci * CHUNK_A
    loc = ci * CHUNK_A
    pltpu.make_async_copy(asd_hbm.at[sbig.at[pl.ds(loc, CHUNK_A)]],
                          arowss[b], sems[2 * b]).wait()
    pltpu.make_async_copy(asd_hbm.at[dbig.at[pl.ds(loc, CHUNK_A)]],
                          browss[b], sems[2 * b + 1]).wait()
    arows, brows, exbuf = arowss[b], browss[b], exbufs[b]
    for k in range(CHUNK_A // 16):
      dadds[b][pl.ds(k * 16, 16)] = dbig[pl.ds(loc + k * 16, 16)]
    if not first:
      pltpu.make_async_copy(exbuf, ex_hbm.at[pl.ds(off, CHUNK_A)],
                            sems[4 + 2 * b]).wait()
    for j in range(CHUNK_A // 16):
      eid = off + j * 16 + iota
      valid = eid < E_TOT
      row = jnp.full((16,), j * 16, jnp.int32) + iota
      for hd in range(HEADS):
        hvec = jnp.full((16,), hd, jnp.int32)
        a = plsc.load_gather(arows, [row, hvec])
        bb = plsc.load_gather(brows, [row, hvec + HEADS])
        al = a + bb
        al = jnp.where(al > 0, al, 0.2 * al)
        ex = jnp.exp(al - cvec[hd])
        ex = jnp.where(valid, ex, 0.0)
        plsc.store_scatter(exbuf, [row, hvec], ex)
    pltpu.async_copy(exbuf, ex_hbm.at[pl.ds(off, CHUNK_A)], sems[4 + 2 * b])
    pltpu.sync_copy(exbuf, den_sp.at[dadds[b]], add=True)

  issue(0, 0)
  issue(1, 1)
  compute(0, 0, True)
  issue(0, 2)
  compute(1, 1, True)

  def pair(g, _):
    i0 = 2 * g
    issue(1, i0 + 1)
    compute(i0, 0, False)
    issue(0, i0 + 2)
    compute(i0 + 1, 1, False)
    return 0

  lax.fori_loop(1, NCHUNK_A // 2 - 1, pair, 0)
  i0 = NCHUNK_A - 2
  issue(1, i0 + 1)
  compute(i0, 0, False)
  compute(i0 + 1, 1, False)
  for b in range(2):
    pltpu.make_async_copy(exbufs[b], ex_hbm.at[pl.ds(base, CHUNK_A)],
                          sems[4 + 2 * b]).wait()

  plsc.subcore_barrier()
  pltpu.sync_copy(den_sp.at[pl.ds(sid * ROWS_PT, ROWS_PT)],
                  den_hbm.at[cid, pl.ds(sid * ROWS_PT, ROWS_PT)])


@functools.partial(
    pl.kernel,
    out_type=(
        jax.ShapeDtypeStruct((EPAD, 16), jnp.float32),
        jax.ShapeDtypeStruct((2, NPAD, 16), jnp.float32),
    ),
    mesh=plsc.VectorSubcoreMesh(core_axis_name="c", subcore_axis_name="s"),
    compiler_params=_SC_PARAMS,
    scratch_types=[
        pltpu.VMEM((16,), jnp.float32),
        pltpu.VMEM((EPW,), jnp.int32),
        pltpu.VMEM((EPW,), jnp.int32),
        [pltpu.VMEM((CHUNK_A,), jnp.int32)] * 2,
        [pltpu.VMEM((CHUNK_A, 16), jnp.float32)] * 2,
        [pltpu.VMEM((CHUNK_A, 16), jnp.float32)] * 2,
        [pltpu.VMEM((CHUNK_A, 16), jnp.float32)] * 2,
        [pltpu.SemaphoreType.DMA] * 8,
        pltpu.VMEM_SHARED((NPAD, 16), jnp.float32),
    ],
)
def _sc_a(s_hbm, d_hbm, asd_hbm, c_hbm, ex_hbm, den_hbm,
          cbuf, sbig, dbig, dadds, exbufs, arowss, browss, sems, den_sp):
    _sca_body(s_hbm, d_hbm, asd_hbm, c_hbm, ex_hbm, den_hbm,
              cbuf, sbig, dbig, dadds, exbufs, arowss, browss, sems, den_sp)


def _scc_body(s_hbm, d_hbm, ex_hbm, dinv_hbm, h_hbm, att_hbm, out_hbm,
              sbufs, dbufs, exbufs, drowss, hrowss, sems, out_sp):
  cid = lax.axis_index("c")
  sid = lax.axis_index("s")
  wid = sid * 2 + cid
  base = wid * EPW

  def zrow(i, _):
    for k in range(HID // 16):
      hrowss[0][i, pl.ds(k * 16, 16)] = jnp.zeros((16,), jnp.float32)
    return 0
  lax.fori_loop(0, CHUNK_C, zrow, 0)
  for b5 in range(ROWS_PT // CHUNK_C):
    pltpu.sync_copy(hrowss[0],
                    out_sp.at[pl.ds(sid * ROWS_PT + b5 * CHUNK_C, CHUNK_C)])
  plsc.subcore_barrier()

  def load_idx(ci, b):
    off = base + ci * CHUNK_C
    pltpu.sync_copy(s_hbm.at[pl.ds(off, CHUNK_C)], sbufs[b])
    pltpu.sync_copy(d_hbm.at[pl.ds(off, CHUNK_C)], dbufs[b])

  def issue(b):
    pltpu.async_copy(h_hbm.at[sbufs[b]], hrowss[b], sems[2 * b])
    pltpu.async_copy(dinv_hbm.at[dbufs[b]], drowss[b], sems[2 * b + 1])

  def compute(ci, b, first):
    off = base + ci * CHUNK_C
    exbuf, drows, hrows = exbufs[b], drowss[b], hrowss[b]
    if not first:
      pltpu.make_async_copy(exbuf, att_hbm.at[pl.ds(off, CHUNK_C)],
                            sems[4 + 2 * b]).wait()
    pltpu.sync_copy(ex_hbm.at[pl.ds(off, CHUNK_C)], exbuf)
    pltpu.make_async_copy(h_hbm.at[sbufs[b]], hrowss[b], sems[2 * b]).wait()
    pltpu.make_async_copy(dinv_hbm.at[dbufs[b]], drowss[b],
                          sems[2 * b + 1]).wait()

    def scale(e, _):
      av = exbuf[e] * drows[e]
      exbuf[e] = av
      for k in range(HID // 16):
        sl = pl.ds(k * 16, 16)
        hrows[e, sl] = hrows[e, sl] * av[k // 2]
      return 0
    lax.fori_loop(0, CHUNK_C, scale, 0)

    pltpu.async_copy(exbuf, att_hbm.at[pl.ds(off, CHUNK_C)], sems[4 + 2 * b])
    pltpu.sync_copy(hrows, out_sp.at[dbufs[b]], add=True)

  load_idx(0, 0)
  issue(0)
  load_idx(1, 1)
  issue(1)
  compute(0, 0, True)
  load_idx(2, 0)
  issue(0)
  compute(1, 1, True)

  def pair(g, _):
    i0 = 2 * g
    load_idx(i0 + 1, 1)
    issue(1)
    compute(i0, 0, False)
    load_idx(i0 + 2, 0)
    issue(0)
    compute(i0 + 1, 1, False)
    return 0

  lax.fori_loop(1, NCHUNK_C // 2 - 1, pair, 0)
  i0 = NCHUNK_C - 2
  load_idx(i0 + 1, 1)
  issue(1)
  compute(i0, 0, False)
  compute(i0 + 1, 1, False)
  for b in range(2):
    pltpu.make_async_copy(exbufs[b], att_hbm.at[pl.ds(base, CHUNK_C)],
                          sems[4 + 2 * b]).wait()

  plsc.subcore_barrier()
  pltpu.sync_copy(out_sp.at[pl.ds(sid * ROWS_PT, ROWS_PT)],
                  out_hbm.at[cid, pl.ds(sid * ROWS_PT, ROWS_PT)])


@functools.partial(
    pl.kernel,
    out_type=(
        jax.ShapeDtypeStruct((EPAD, 16), jnp.float32),
        jax.ShapeDtypeStruct((2, NPAD, HID), jnp.float32),
    ),
    mesh=plsc.VectorSubcoreMesh(core_axis_name="c", subcore_axis_name="s"),
    compiler_params=_SC_PARAMS,
    scratch_types=[
        [pltpu.VMEM((CHUNK_C,), jnp.int32)] * 2,
        [pltpu.VMEM((CHUNK_C,), jnp.int32)] * 2,
        [pltpu.VMEM((CHUNK_C, 16), jnp.float32)] * 2,
        [pltpu.VMEM((CHUNK_C, 16), jnp.float32)] * 2,
        [pltpu.VMEM((CHUNK_C, HID), jnp.float32)] * 2,
        [pltpu.SemaphoreType.DMA] * 8,
        pltpu.VMEM_SHARED((NPAD, HID), jnp.float32),
    ],
)
def _sc_c(s_hbm, d_hbm, ex_hbm, dinv_hbm, h_hbm, att_hbm, out_hbm,
          sbufs, dbufs, exbufs, drowss, hrowss, sems, out_sp):
    _scc_body(s_hbm, d_hbm, ex_hbm, dinv_hbm, h_hbm, att_hbm, out_hbm,
              sbufs, dbufs, exbufs, drowss, hrowss, sems, out_sp)


_SEL = np.zeros((HID, HEADS), np.float32)
for _i in range(HID):
    _SEL[_i, _i // C_HEAD] = 1.0


def _mk_m(a_src, a_dst):
    return jnp.concatenate(
        [a_src.reshape(HID, 1) * _SEL, a_dst.reshape(HID, 1) * _SEL,
         jnp.zeros((HID, 6), jnp.float32)], axis=1)


def kernel(x, edge_index, batch, W1, a_src1, a_dst1, b1, W2, a_src2, a_dst2,
           b2, g1, be1, g2, be2, Wm, bm, Wr1, br1, gr, ber, Wr2, br2):
    loop = jnp.arange(N, dtype=edge_index.dtype)
    s = jnp.concatenate([edge_index[0], loop])
    d = jnp.concatenate([edge_index[1], loop])
    s = jnp.pad(s, (0, EPAD - E_TOT))
    d = jnp.pad(d, (0, EPAD - E_TOT))

    m1 = _mk_m(a_src1, a_dst1)
    m2 = _mk_m(a_src2, a_dst2)
    oh = (jnp.arange(G, dtype=batch.dtype)[:, None] == batch[None, :]
          ).astype(jnp.float32)

    h1r, asd1, c1 = _tc_pre(x, W1, m1)
    ex1, den1 = _sc_a(s, d, asd1, c1)
    dinv1 = _tc_dinv(den1)
    att1, agg1 = _sc_c(s, d, ex1, dinv1, h1r)

    h2r, asd2, c2 = _tc_mid(agg1, b1, g1, be1, W2, m2)
    ex2, den2 = _sc_a(s, d, asd2, c2)
    dinv2 = _tc_dinv(den2)
    att2, agg2 = _sc_c(s, d, ex2, dinv2, h2r)

    xo, rec = _tc_final(agg2, b2, g2, be2, Wm, bm, oh,
                        Wr1, br1, gr, ber, Wr2, br2)
    return (xo, rec, att1[:E_TOT, :HEADS], att2[:E_TOT, :HEADS])
